# Initial kernel scaffold; baseline (speedup 1.0000x reference)
import jax
import jax.numpy as jnp
from jax.experimental import pallas as pl


def kernel(samples, bags_num_samples):
    raise NotImplementedError("write your pallas kernel here")


# speedup vs baseline: 1.0635x; 1.0635x over previous
import jax
import jax.numpy as jnp
from jax import lax
from jax.experimental import pallas as pl
from jax.experimental.pallas import tpu as pltpu
from jax.experimental.pallas import tpu_sc as plsc

N_ROWS = 32768
D = 1024
N_BAGS = 16
L = 16
HALF = D // 2
R = 64
JGROUPS = HALF // L


def _body(samples_hbm, counts_hbm, starts_hbm, out_hbm, counts_v, starts_v,
          buf, acc):
    c = lax.axis_index("c")
    s = lax.axis_index("s")
    bag = s
    col0 = c * HALF

    pltpu.sync_copy(counts_hbm, counts_v)
    pltpu.sync_copy(starts_hbm, starts_v)
    bag_v = jnp.full((L,), bag, jnp.int32)
    count = plsc.load_gather(counts_v, [bag_v])[0]
    start = plsc.load_gather(starts_v, [bag_v])[0]
    end = start + count

    for j in range(JGROUPS):
        acc[pl.ds(L * j, L)] = jnp.zeros((L,), jnp.float32)

    astart = (start // 8) * 8
    n_chunks = lax.div(end - astart + (R - 1), R)

    def chunk_body(g, _):
        cbase = astart + g * R
        base = pl.multiple_of(jnp.minimum(cbase, N_ROWS - R), 8)
        pltpu.sync_copy(
            samples_hbm.at[pl.ds(base, R), pl.ds(col0, HALF)], buf)
        lo = jnp.maximum(start, cbase) - base
        hi = jnp.minimum(end, cbase + R) - base

        for j in range(JGROUPS):
            def rbody(r, a, j=j):
                return a + buf[r, pl.ds(L * j, L)]
            acc[pl.ds(L * j, L)] = lax.fori_loop(
                lo, hi, rbody, acc[pl.ds(L * j, L)])
        return 0

    lax.fori_loop(0, n_chunks, chunk_body, 0)

    cnt_v = jnp.full((L,), count, jnp.int32).astype(jnp.float32)
    for j in range(JGROUPS):
        acc[pl.ds(L * j, L)] = acc[pl.ds(L * j, L)] / cnt_v
    out_off = pl.multiple_of(bag * D + col0, HALF)
    pltpu.sync_copy(acc, out_hbm.at[pl.ds(out_off, HALF)])


@jax.jit
def kernel(samples, bags_num_samples):
    mesh = plsc.VectorSubcoreMesh(core_axis_name="c", subcore_axis_name="s")
    run = pl.kernel(
        _body,
        out_type=jax.ShapeDtypeStruct((N_BAGS * D,), jnp.float32),
        mesh=mesh,
        compiler_params=pltpu.CompilerParams(needs_layout_passes=False),
        scratch_types=[
            pltpu.VMEM((L,), jnp.int32),
            pltpu.VMEM((L,), jnp.int32),
            pltpu.VMEM((R, HALF), jnp.float32),
            pltpu.VMEM((HALF,), jnp.float32),
        ],
    )
    starts = jnp.cumsum(bags_num_samples) - bags_num_samples
    return run(samples, bags_num_samples, starts).reshape(N_BAGS, D)


# speedup vs baseline: 3.0381x; 2.8568x over previous
import jax
import jax.numpy as jnp
from jax import lax
from jax.experimental import pallas as pl
from jax.experimental.pallas import tpu as pltpu
from jax.experimental.pallas import tpu_sc as plsc

N_ROWS = 32768
D = 1024
N_BAGS = 16
L = 16
HALF = D // 2
R = 64
JGROUPS = HALF // L


def _tree_sum(vals):
    while len(vals) > 1:
        vals = [vals[i] + vals[i + 1] for i in range(0, len(vals) - 1, 2)] + (
            [vals[-1]] if len(vals) % 2 else [])
    return vals[0]


def _body(samples_hbm, counts_hbm, starts_hbm, out_hbm, counts_v, starts_v,
          buf, acc):
    c = lax.axis_index("c")
    s = lax.axis_index("s")
    bag = s
    col0 = c * HALF

    pltpu.sync_copy(counts_hbm, counts_v)
    pltpu.sync_copy(starts_hbm, starts_v)
    bag_v = jnp.full((L,), bag, jnp.int32)
    count = plsc.load_gather(counts_v, [bag_v])[0]
    start = plsc.load_gather(starts_v, [bag_v])[0]
    end = start + count

    for j in range(JGROUPS):
        acc[pl.ds(L * j, L)] = jnp.zeros((L,), jnp.float32)

    astart = (start // 8) * 8
    n_chunks = lax.div(end - astart + (R - 1), R)
    zero_row = jnp.zeros((L,), jnp.float32)

    def chunk_body(g, _):
        cbase = astart + g * R
        base = pl.multiple_of(jnp.minimum(cbase, N_ROWS - R), 8)
        pltpu.sync_copy(
            samples_hbm.at[pl.ds(base, R), pl.ds(col0, HALF)], buf)
        lo = jnp.maximum(start, cbase) - base
        hi = jnp.minimum(end, cbase + R) - base

        def zero_one(r, _):
            def zj(j, _):
                buf[r, pl.ds(L * j, L)] = zero_row
                return 0
            lax.fori_loop(0, JGROUPS, zj, 0)
            return 0

        lax.fori_loop(0, lo, zero_one, 0)
        lax.fori_loop(hi, R, zero_one, 0)

        @plsc.parallel_loop(0, JGROUPS)
        def jstep(j):
            off = pl.ds(L * j, L)
            acc[off] = acc[off] + _tree_sum([buf[r, off] for r in range(R)])

        return 0

    lax.fori_loop(0, n_chunks, chunk_body, 0)

    cnt_v = jnp.full((L,), count, jnp.int32).astype(jnp.float32)
    for j in range(JGROUPS):
        acc[pl.ds(L * j, L)] = acc[pl.ds(L * j, L)] / cnt_v
    out_off = pl.multiple_of(bag * D + col0, HALF)
    pltpu.sync_copy(acc, out_hbm.at[pl.ds(out_off, HALF)])


@jax.jit
def kernel(samples, bags_num_samples):
    mesh = plsc.VectorSubcoreMesh(core_axis_name="c", subcore_axis_name="s")
    run = pl.kernel(
        _body,
        out_type=jax.ShapeDtypeStruct((N_BAGS * D,), jnp.float32),
        mesh=mesh,
        compiler_params=pltpu.CompilerParams(needs_layout_passes=False),
        scratch_types=[
            pltpu.VMEM((L,), jnp.int32),
            pltpu.VMEM((L,), jnp.int32),
            pltpu.VMEM((R, HALF), jnp.float32),
            pltpu.VMEM((HALF,), jnp.float32),
        ],
    )
    starts = jnp.cumsum(bags_num_samples) - bags_num_samples
    return run(samples, bags_num_samples, starts).reshape(N_BAGS, D)


# speedup vs baseline: 4.2437x; 1.3968x over previous
import jax
import jax.numpy as jnp
from jax import lax
from jax.experimental import pallas as pl
from jax.experimental.pallas import tpu as pltpu
from jax.experimental.pallas import tpu_sc as plsc

N_ROWS = 32768
D = 1024
N_BAGS = 16
L = 16
HALF = D // 2
R = 64
JGROUPS = HALF // L


def _tree_sum(vals):
    while len(vals) > 1:
        vals = [vals[i] + vals[i + 1] for i in range(0, len(vals) - 1, 2)] + (
            [vals[-1]] if len(vals) % 2 else [])
    return vals[0]


def _body(samples_hbm, counts_hbm, starts_hbm, out_hbm, counts_v, starts_v,
          buf0, buf1, acc, sem0, sem1):
    c = lax.axis_index("c")
    s = lax.axis_index("s")
    bag = s
    col0 = c * HALF
    bufs = (buf0, buf1)
    sems = (sem0, sem1)

    pltpu.sync_copy(counts_hbm, counts_v)
    pltpu.sync_copy(starts_hbm, starts_v)
    bag_v = jnp.full((L,), bag, jnp.int32)
    count = plsc.load_gather(counts_v, [bag_v])[0]
    start = plsc.load_gather(starts_v, [bag_v])[0]
    end = start + count

    for j in range(JGROUPS):
        acc[pl.ds(L * j, L)] = jnp.zeros((L,), jnp.float32)

    astart = (start // 8) * 8
    n_chunks = lax.div(end - astart + (R - 1), R)
    zero_row = jnp.zeros((L,), jnp.float32)

    def chunk_base(g):
        return pl.multiple_of(
            jnp.minimum(astart + g * R, N_ROWS - R), 8)

    def start_dma(g, b):
        pltpu.async_copy(
            samples_hbm.at[pl.ds(chunk_base(g), R), pl.ds(col0, HALF)],
            bufs[b], sems[b])

    def wait_dma(b):
        pltpu.make_async_copy(
            samples_hbm.at[pl.ds(0, R), pl.ds(col0, HALF)],
            bufs[b], sems[b]).wait()

    def compute(g, b):
        buf = bufs[b]
        cbase = astart + g * R
        base = chunk_base(g)
        lo = jnp.maximum(start, cbase) - base
        hi = jnp.minimum(end, cbase + R) - base

        def zero_one(r, _):
            def zj(j, _):
                buf[r, pl.ds(L * j, L)] = zero_row
                return 0
            lax.fori_loop(0, JGROUPS, zj, 0)
            return 0

        lax.fori_loop(0, lo, zero_one, 0)
        lax.fori_loop(hi, R, zero_one, 0)

        @plsc.parallel_loop(0, JGROUPS)
        def jstep(j):
            off = pl.ds(L * j, L)
            acc[off] = acc[off] + _tree_sum([buf[r, off] for r in range(R)])

    start_dma(0, 0)

    def pair_body(i, _):
        g2 = i * 2
        for b in range(2):
            g = g2 + b

            @pl.when(g < n_chunks)
            def _():
                wait_dma(b)

                @pl.when(g + 1 < n_chunks)
                def _():
                    start_dma(g + 1, 1 - b)

                compute(g, b)
        return 0

    lax.fori_loop(0, (n_chunks + 1) // 2, pair_body, 0)

    cnt_v = jnp.full((L,), count, jnp.int32).astype(jnp.float32)
    for j in range(JGROUPS):
        acc[pl.ds(L * j, L)] = acc[pl.ds(L * j, L)] / cnt_v
    out_off = pl.multiple_of(bag * D + col0, HALF)
    pltpu.sync_copy(acc, out_hbm.at[pl.ds(out_off, HALF)])


@jax.jit
def kernel(samples, bags_num_samples):
    mesh = plsc.VectorSubcoreMesh(core_axis_name="c", subcore_axis_name="s")
    run = pl.kernel(
        _body,
        out_type=jax.ShapeDtypeStruct((N_BAGS * D,), jnp.float32),
        mesh=mesh,
        compiler_params=pltpu.CompilerParams(needs_layout_passes=False),
        scratch_types=[
            pltpu.VMEM((L,), jnp.int32),
            pltpu.VMEM((L,), jnp.int32),
            pltpu.VMEM((R, HALF), jnp.float32),
            pltpu.VMEM((R, HALF), jnp.float32),
            pltpu.VMEM((HALF,), jnp.float32),
            pltpu.SemaphoreType.DMA,
            pltpu.SemaphoreType.DMA,
        ],
    )
    starts = jnp.cumsum(bags_num_samples) - bags_num_samples
    return run(samples, bags_num_samples, starts).reshape(N_BAGS, D)
